# baseline (speedup 1.0000x reference)
import functools

import jax
import jax.numpy as jnp
from jax import lax
from jax.experimental import pallas as pl
from jax.experimental.pallas import tpu as pltpu
from jax.experimental.pallas import tpu_sc as plsc

VOCAB_SIZE = 1000000
EMBED_SIZE = 64
BATCH = 16384
SEQ_LEN = 200

_INFO = plsc.get_sparse_core_info()
NC = _INFO.num_cores
NS = _INFO.num_subcores
NW = NC * NS
LANE = 128

NBLK = (BATCH // LANE) * SEQ_LEN
BLK_PER_W = NBLK // NW
SCH = 8
NSCH = BLK_PER_W // SCH


def _sc_gather(idx_t, table):
    mesh = plsc.VectorSubcoreMesh(core_axis_name="c", subcore_axis_name="s")

    @functools.partial(
        pl.kernel,
        mesh=mesh,
        out_type=jax.ShapeDtypeStruct((SEQ_LEN, 8, 128, 8, 128), jnp.float32),
        scratch_types=[
            pltpu.VMEM((SCH, LANE), jnp.int32),
            pltpu.VMEM((SCH * LANE, EMBED_SIZE), jnp.float32),
            pltpu.VMEM((8, 8, 129), jnp.float32),
            pltpu.VMEM((8, 8, 129), jnp.float32),
            pltpu.VMEM((8, 8, 128), jnp.float32),
            pltpu.VMEM((8, 8, 128), jnp.float32),
            pltpu.SemaphoreType.DMA,
            pltpu.SemaphoreType.DMA,
        ],
        compiler_params=pltpu.CompilerParams(
            use_tc_tiling_on_sc=False,
            needs_layout_passes=False,
            disable_bounds_checks=True,
        ),
    )
    def body(
        idx_hbm, table_hbm, out_hbm,
        idx_v, rows_v, tiles_a, tiles_b, wbuf_a, wbuf_b, gsem, osem,
    ):
        wid = lax.axis_index("s") * NC + lax.axis_index("c")
        base = wid * BLK_PER_W
        lanes = lax.iota(jnp.int32, 16)
        dhi_c = [(16 * k + lanes) // 8 for k in range(4)]
        dlo_c = [lax.rem(16 * k + lanes, 8) for k in range(4)]
        tiles_refs = (tiles_a, tiles_b)
        wbuf_refs = (wbuf_a, wbuf_b)

        def superchunk(i, n):
            b0 = base + i * SCH
            pltpu.sync_copy(idx_hbm.at[pl.ds(b0, SCH)], idx_v)
            for j in range(SCH):
                pltpu.async_copy(
                    table_hbm.at[idx_v.at[j]],
                    rows_v.at[pl.ds(j * LANE, LANE)],
                    gsem,
                )

            def pair(p, m):
                for q in range(2):
                    j = p * 2 + q
                    ncur = m + q
                    beta = b0 + j
                    s = beta // 128
                    J = lax.rem(beta, 128)
                    pltpu.make_async_copy(
                        table_hbm.at[idx_v.at[0]],
                        rows_v.at[pl.ds(0, LANE)],
                        gsem,
                    ).wait()

                    @pl.when(ncur >= 2)
                    def _():
                        pltpu.make_async_copy(
                            wbuf_a, out_hbm.at[0, :, 0], osem
                        ).wait()

                    row0 = j * LANE
                    tl = tiles_refs[q]
                    rblk = rows_v.at[pl.ds(row0, LANE)]

                    @plsc.parallel_loop(0, LANE, unroll=8)
                    def tstep(t):
                        col_t = jnp.full((16,), t, jnp.int32)
                        for k in range(4):
                            v = rblk[t, pl.ds(k * 16, 16)]
                            plsc.store_scatter(
                                tl, [dhi_c[k], dlo_c[k], col_t], v
                            )

                    wb = wbuf_refs[q]

                    @plsc.parallel_loop(0, 64, unroll=8)
                    def cstep(r):
                        rhi = r // 8
                        rlo = lax.rem(r, 8)
                        for g in range(8):
                            wb[rhi, rlo, pl.ds(g * 16, 16)] = tl[
                                rhi, rlo, pl.ds(g * 16, 16)
                            ]

                    pltpu.async_copy(wb, out_hbm.at[s, :, J], osem)
                return m + 2

            return lax.fori_loop(0, SCH // 2, pair, n)

        lax.fori_loop(0, NSCH, superchunk, 0)

        for _ in range(2):
            pltpu.make_async_copy(wbuf_a, out_hbm.at[0, :, 0], osem).wait()

    return body(idx_t, table)


def kernel(input, embeddings):
    idx_t = jnp.reshape(
        jnp.transpose(input.astype(jnp.int32)), (NBLK, LANE)
    )
    out5 = _sc_gather(idx_t, embeddings)
    out = jnp.transpose(out5, (2, 4, 0, 1, 3))
    return jnp.reshape(out, (BATCH, SEQ_LEN, EMBED_SIZE))


# speedup vs baseline: 1.1446x; 1.1446x over previous
import functools

import jax
import jax.numpy as jnp
from jax import lax
from jax.experimental import pallas as pl
from jax.experimental.pallas import tpu as pltpu
from jax.experimental.pallas import tpu_sc as plsc

VOCAB_SIZE = 1000000
EMBED_SIZE = 64
BATCH = 16384
SEQ_LEN = 200

_INFO = plsc.get_sparse_core_info()
NC = _INFO.num_cores
NS = _INFO.num_subcores
NW = NC * NS
LANE = 128

JPW = (BATCH // LANE) // NW
SGRP = 8
NGRP = SEQ_LEN // SGRP
BPG = SGRP * JPW
RING = 8


def _sc_gather(idx4, table):
    mesh = plsc.VectorSubcoreMesh(core_axis_name="c", subcore_axis_name="s")

    @functools.partial(
        pl.kernel,
        mesh=mesh,
        out_type=jax.ShapeDtypeStruct((SEQ_LEN, 8, 128, 8, 128), jnp.float32),
        scratch_types=[
            pltpu.VMEM((JPW, SGRP, LANE), jnp.int32),
            pltpu.VMEM((RING * LANE, EMBED_SIZE), jnp.float32),
            pltpu.VMEM((8, 8, 129), jnp.float32),
            pltpu.VMEM((8, 8, 129), jnp.float32),
            pltpu.SemaphoreType.DMA,
            pltpu.SemaphoreType.DMA,
        ],
        compiler_params=pltpu.CompilerParams(
            use_tc_tiling_on_sc=False,
            needs_layout_passes=False,
            disable_bounds_checks=True,
        ),
    )
    def body(idx_hbm, table_hbm, out_hbm, idx_v, rows_v, tiles_a, tiles_b, gsem, osem):
        wid = lax.axis_index("s") * NC + lax.axis_index("c")
        j0 = wid * JPW
        lanes = lax.iota(jnp.int32, 16)
        dhi_c = [(16 * k + lanes) // 8 for k in range(4)]
        dlo_c = [lax.rem(16 * k + lanes, 8) for k in range(4)]
        tiles_refs = (tiles_a, tiles_b)

        def fire(n):
            jq = lax.rem(n, JPW)
            h = n // JPW
            slot = lax.rem(n, RING)
            pltpu.async_copy(
                table_hbm.at[idx_v.at[jq, h]],
                rows_v.at[pl.ds(slot * LANE, LANE)],
                gsem,
            )

        def group(m, cnt):
            pltpu.sync_copy(idx_hbm.at[m, pl.ds(j0, JPW)], idx_v)
            for n in range(RING):
                fire(n)

            def pair(p, c2):
                for q in range(2):
                    n = p * 2 + q
                    ncur = c2 + q
                    jq = lax.rem(n, JPW)
                    h = n // JPW
                    s = m * SGRP + h
                    J = j0 + jq
                    pltpu.make_async_copy(
                        table_hbm.at[idx_v.at[0, 0]],
                        rows_v.at[pl.ds(0, LANE)],
                        gsem,
                    ).wait()

                    @pl.when(ncur >= 2)
                    def _():
                        pltpu.make_async_copy(
                            tiles_a.at[:, :, pl.ds(0, 128)],
                            out_hbm.at[0, :, 0],
                            osem,
                        ).wait()

                    slot = lax.rem(n, RING)
                    tl = tiles_refs[q]
                    rblk = rows_v.at[pl.ds(slot * LANE, LANE)]

                    @plsc.parallel_loop(0, LANE, unroll=8)
                    def tstep(t):
                        col_t = jnp.full((16,), t, jnp.int32)
                        for k in range(4):
                            v = rblk[t, pl.ds(k * 16, 16)]
                            plsc.store_scatter(
                                tl, [dhi_c[k], dlo_c[k], col_t], v
                            )

                    pltpu.async_copy(
                        tl.at[:, :, pl.ds(0, 128)], out_hbm.at[s, :, J], osem
                    )

                    @pl.when(n < BPG - RING)
                    def _():
                        fire(n + RING)
                return c2 + 2

            return lax.fori_loop(0, BPG // 2, pair, cnt)

        lax.fori_loop(0, NGRP, group, 0)

        for _ in range(2):
            pltpu.make_async_copy(
                tiles_a.at[:, :, pl.ds(0, 128)], out_hbm.at[0, :, 0], osem
            ).wait()

    return body(idx4, table)


def kernel(input, embeddings):
    idx4 = jnp.transpose(
        jnp.reshape(input.astype(jnp.int32), (128, 128, 25, 8)),
        (2, 0, 3, 1),
    )
    out5 = _sc_gather(idx4, embeddings)
    out = jnp.transpose(out5, (2, 4, 0, 1, 3))
    return jnp.reshape(out, (BATCH, SEQ_LEN, EMBED_SIZE))
